# Initial kernel scaffold; baseline (speedup 1.0000x reference)
import jax
import jax.numpy as jnp
from jax.experimental import pallas as pl


def kernel(x, edge_index, W1, b1, W2, b2, W3, b3, W4, b4, W5, b5, W6, b6):
    raise NotImplementedError("write your pallas kernel here")


# speedup vs baseline: 1.0253x; 1.0253x over previous
import functools

import jax
import jax.numpy as jnp
from jax.experimental import pallas as pl


def _mm_kernel(x_ref, w_ref, o_ref):
    o_ref[...] = jax.lax.dot_general(
        x_ref[...], w_ref[...], (((1,), (0,)), ((), ())),
        preferred_element_type=jnp.float32,
        precision=jax.lax.Precision.HIGHEST,
    )


@functools.partial(jax.jit, static_argnames=())
def _matmul(x, w):
    m, k = x.shape
    k2, n = w.shape
    bm = 2000
    n_pad = max(128, ((n + 127) // 128) * 128)
    if n_pad != n:
        w = jnp.pad(w, ((0, 0), (0, n_pad - n)))
    out = pl.pallas_call(
        _mm_kernel,
        grid=(m // bm,),
        in_specs=[
            pl.BlockSpec((bm, k), lambda i: (i, 0)),
            pl.BlockSpec((k, n_pad), lambda i: (0, 0)),
        ],
        out_specs=pl.BlockSpec((bm, n_pad), lambda i: (i, 0)),
        out_shape=jax.ShapeDtypeStruct((m, n_pad), jnp.float32),
    )(x, w)
    if n_pad != n:
        out = out[:, :n]
    return out


def kernel(x, edge_index, W1, b1, W2, b2, W3, b3, W4, b4, W5, b5, W6, b6):
    src, dst = edge_index[0], edge_index[1]
    n = x.shape[0]
    loop = jnp.arange(n, dtype=src.dtype)
    s = jnp.concatenate([src, loop])
    d = jnp.concatenate([dst, loop])
    deg = jax.ops.segment_sum(jnp.ones(s.shape[0], dtype=x.dtype), d, num_segments=n)
    dinv = jax.lax.rsqrt(jnp.maximum(deg, 1e-12))
    norm = dinv[s] * dinv[d]

    h = x
    for W, b, act in (
        (W1, b1, True), (W2, b2, True), (W3, b3, True),
        (W4, b4, True), (W5, b5, True), (W6, b6, False),
    ):
        hw = _matmul(h, W)
        agg = jax.ops.segment_sum(hw[s] * norm[:, None], d, num_segments=n)
        h = agg + b
        if act:
            h = jax.nn.relu(h)
    return jax.nn.log_softmax(h, axis=1)


# speedup vs baseline: 6.2986x; 6.1432x over previous
import functools

import jax
import jax.numpy as jnp
from jax import lax
from jax.experimental import pallas as pl
from jax.experimental.pallas import tpu as pltpu
from jax.experimental.pallas import tpu_sc as plsc

N_NODES = 10000
NPAD = 10240
N_EDGES = 320000
NB = 79
EB = 128
EP = 32 * NB * EB
TRASH = 10000
ROWS_PER_SUB = NPAD // 16

_mesh = plsc.VectorSubcoreMesh(core_axis_name="c", subcore_axis_name="s")


@functools.partial(
    pl.kernel,
    out_type=jax.ShapeDtypeStruct((2, NPAD, 128), jnp.float32),
    mesh=_mesh,
    scratch_types=[
        pltpu.VMEM((NB, EB), jnp.int32),
        pltpu.VMEM((NB, EB), jnp.int32),
        pltpu.VMEM((EB, 128), jnp.float32),
        pltpu.VMEM_SHARED((NPAD, 128), jnp.float32),
        pltpu.SemaphoreType.DMA,
    ],
)
def _sc_aggregate(table_hbm, src_hbm, dst_hbm, zeros_hbm, out_hbm,
                  src_v, dst_v, buf, acc, sem):
    c = lax.axis_index("c")
    s = lax.axis_index("s")
    wid = s * 2 + c
    pltpu.sync_copy(src_hbm.at[wid], src_v)
    pltpu.sync_copy(dst_hbm.at[wid], dst_v)
    pltpu.sync_copy(zeros_hbm, acc.at[pl.ds(s * ROWS_PER_SUB, ROWS_PER_SUB)])
    plsc.subcore_barrier()

    def body(b, carry):
        pltpu.async_copy(table_hbm.at[src_v.at[b]], buf, sem).wait()
        pltpu.sync_copy(buf, acc.at[dst_v.at[b]], add=True)
        return carry

    lax.fori_loop(0, NB, body, 0)
    plsc.subcore_barrier()
    pltpu.sync_copy(
        acc.at[pl.ds(s * ROWS_PER_SUB, ROWS_PER_SUB)],
        out_hbm.at[c].at[pl.ds(s * ROWS_PER_SUB, ROWS_PER_SUB)],
    )


def _mm_body(x_ref, w_ref, o_ref):
    o_ref[...] = jax.lax.dot_general(
        x_ref[...], w_ref[...], (((1,), (0,)), ((), ())),
        preferred_element_type=jnp.float32,
        precision=jax.lax.Precision.HIGHEST,
    )


def _matmul(x, w):
    m, k = x.shape
    _, n = w.shape
    bm = 2000
    n_pad = ((n + 127) // 128) * 128
    if n_pad != n:
        w = jnp.pad(w, ((0, 0), (0, n_pad - n)))
    return pl.pallas_call(
        _mm_body,
        grid=(m // bm,),
        in_specs=[
            pl.BlockSpec((bm, k), lambda i: (i, 0)),
            pl.BlockSpec((k, n_pad), lambda i: (0, 0)),
        ],
        out_specs=pl.BlockSpec((bm, n_pad), lambda i: (i, 0)),
        out_shape=jax.ShapeDtypeStruct((m, n_pad), jnp.float32),
    )(x, w)


def kernel(x, edge_index, W1, b1, W2, b2, W3, b3, W4, b4, W5, b5, W6, b6):
    src = edge_index[0].astype(jnp.int32)
    dst = edge_index[1].astype(jnp.int32)
    pad = EP - N_EDGES
    src_p = jnp.concatenate([src, jnp.zeros((pad,), jnp.int32)]).reshape(32, NB, EB)
    dst_p = jnp.concatenate([dst, jnp.full((pad,), TRASH, jnp.int32)]).reshape(32, NB, EB)

    zeros128 = jnp.zeros((ROWS_PER_SUB, 128), jnp.float32)

    ones_tab = jnp.ones((N_NODES, 128), jnp.float32)
    dpart = _sc_aggregate(ones_tab, src_p, dst_p, zeros128)
    deg = dpart[0, :N_NODES, 0] + dpart[1, :N_NODES, 0] + 1.0
    dinv = lax.rsqrt(jnp.maximum(deg, 1e-12))[:, None]

    def aggregate(hs):
        f = hs.shape[1]
        outs = []
        for ci in range(f // 128):
            part = _sc_aggregate(hs[:, ci * 128:(ci + 1) * 128], src_p, dst_p, zeros128)
            outs.append(part[0, :N_NODES] + part[1, :N_NODES])
        return outs[0] if len(outs) == 1 else jnp.concatenate(outs, axis=1)

    xs = dinv * x
    u = dinv * (aggregate(xs) + xs)
    h = jax.nn.relu(_matmul(u, W1)[:, :640] + b1)

    for W, b, act in ((W2, b2, True), (W3, b3, True), (W4, b4, True),
                      (W5, b5, True), (W6, b6, False)):
        n_out = W.shape[1]
        t = _matmul(h, W)
        hs = dinv * t
        h = dinv * (aggregate(hs) + hs)[:, :n_out] + b
        if act:
            h = jax.nn.relu(h)
    return jax.nn.log_softmax(h, axis=1)
